# Initial kernel scaffold; baseline (speedup 1.0000x reference)
import jax
import jax.numpy as jnp
from jax.experimental import pallas as pl


def kernel(labels, weight):
    raise NotImplementedError("write your pallas kernel here")


# speedup vs baseline: 1.0939x; 1.0939x over previous
import functools

import jax
import jax.numpy as jnp
from jax import lax
from jax.experimental import pallas as pl
from jax.experimental.pallas import tpu as pltpu
from jax.experimental.pallas import tpu_sc as plsc

NUM_EMB = 1_000_000
DIM = 32
BATCH = 16384
HIST = 50
B = BATCH * HIST

NC = 2
NS = 16
NW = NC * NS
B_PER_W = B // NW

IDXW = 128
KG = 8
BLK = KG * IDXW
N_BLK = B_PER_W // BLK
LROWS = B // IDXW

_mesh = plsc.VectorSubcoreMesh(core_axis_name="c", subcore_axis_name="s")


@functools.partial(
    pl.kernel,
    mesh=_mesh,
    out_type=jax.ShapeDtypeStruct((B, DIM), jnp.float32),
    scratch_types=[
        pltpu.VMEM((KG, IDXW), jnp.int32),
        pltpu.VMEM((BLK, DIM), jnp.float32),
        pltpu.SemaphoreType.DMA,
        pltpu.SemaphoreType.DMA,
    ],
    compiler_params=pltpu.CompilerParams(use_tc_tiling_on_sc=False),
)
def _gather_kernel(lab_hbm, w_hbm, out_hbm, idx_v, rows_v, sem_i, sem_g):
    wid = lax.axis_index("s") * NC + lax.axis_index("c")
    row0 = wid * B_PER_W
    grp0 = wid * (B_PER_W // IDXW)

    def body(g, carry):
        pltpu.async_copy(lab_hbm.at[pl.ds(grp0 + g * KG, KG)], idx_v, sem_i).wait()
        cps = [
            pltpu.async_copy(
                w_hbm.at[idx_v.at[j]],
                rows_v.at[pl.ds(j * IDXW, IDXW)],
                sem_g,
            )
            for j in range(KG)
        ]
        for c in cps:
            c.wait()
        pltpu.async_copy(rows_v, out_hbm.at[pl.ds(row0 + g * BLK, BLK)], sem_i).wait()
        return carry

    lax.fori_loop(0, N_BLK, body, 0)


def kernel(labels, weight):
    lab2d = labels.astype(jnp.int32).reshape(LROWS, IDXW)
    out = _gather_kernel(lab2d, weight)
    return out.reshape(BATCH, HIST, DIM)


# speedup vs baseline: 1.1054x; 1.0105x over previous
import functools

import jax
import jax.numpy as jnp
from jax import lax
from jax.experimental import pallas as pl
from jax.experimental.pallas import tpu as pltpu
from jax.experimental.pallas import tpu_sc as plsc

NUM_EMB = 1_000_000
DIM = 32
BATCH = 16384
HIST = 50
B = BATCH * HIST

NC = 2
NS = 16
NW = NC * NS
B_PER_W = B // NW

IDXW = 128
KG = 8
BLK = KG * IDXW
N_BLK = B_PER_W // BLK
N_PAIR = (N_BLK - 1) // 2
LROWS = B // IDXW

_mesh = plsc.VectorSubcoreMesh(core_axis_name="c", subcore_axis_name="s")


@functools.partial(
    pl.kernel,
    mesh=_mesh,
    out_type=jax.ShapeDtypeStruct((B, DIM), jnp.float32),
    scratch_types=[
        pltpu.VMEM((2, KG, IDXW), jnp.int32),
        pltpu.VMEM((2, BLK, DIM), jnp.float32),
        pltpu.SemaphoreType.DMA,
        pltpu.SemaphoreType.DMA,
        pltpu.SemaphoreType.DMA,
        pltpu.SemaphoreType.DMA,
        pltpu.SemaphoreType.DMA,
    ],
    compiler_params=pltpu.CompilerParams(use_tc_tiling_on_sc=False),
)
def _gather_kernel(lab_hbm, w_hbm, out_hbm, idx_v, rows_v, sem_i,
                   sem_g0, sem_g1, sem_o0, sem_o1):
    wid = lax.axis_index("s") * NC + lax.axis_index("c")
    row0 = wid * B_PER_W
    grp0 = wid * (B_PER_W // IDXW)
    sem_g = (sem_g0, sem_g1)
    sem_o = (sem_o0, sem_o1)

    def stage_idx(b, u):
        pltpu.async_copy(lab_hbm.at[pl.ds(grp0 + b * KG, KG)],
                         idx_v.at[u], sem_i).wait()

    def fire_gathers(u):
        for j in range(KG):
            pltpu.async_copy(
                w_hbm.at[idx_v.at[u, j]],
                rows_v.at[u, pl.ds(j * IDXW, IDXW)],
                sem_g[u],
            )

    def wait_gathers(u):
        for j in range(KG):
            pltpu.make_async_copy(
                w_hbm.at[idx_v.at[u, j]],
                rows_v.at[u, pl.ds(j * IDXW, IDXW)],
                sem_g[u],
            ).wait()

    def fire_out(b, u):
        pltpu.async_copy(rows_v.at[u],
                         out_hbm.at[pl.ds(row0 + b * BLK, BLK)], sem_o[u])

    def wait_out(b, u):
        pltpu.make_async_copy(rows_v.at[u],
                              out_hbm.at[pl.ds(row0 + b * BLK, BLK)],
                              sem_o[u]).wait()

    stage_idx(0, 0)
    fire_gathers(0)

    def body(p, carry):
        for u in (0, 1):
            v = 1 - u
            b = 2 * p + u
            wait_gathers(u)
            fire_out(b, u)

            @pl.when(b + 1 < N_BLK)
            def _():
                stage_idx(b + 1, v)

            @pl.when(b >= 1)
            def _():
                wait_out(b - 1, v)

            @pl.when(b + 1 < N_BLK)
            def _():
                fire_gathers(v)

        return carry

    lax.fori_loop(0, N_PAIR, body, 0)
    last = N_BLK - 1
    wait_gathers(last % 2)
    fire_out(last, last % 2)
    wait_out(last - 1, 1 - (last % 2))
    wait_out(last, last % 2)


def kernel(labels, weight):
    lab2d = labels.astype(jnp.int32).reshape(LROWS, IDXW)
    out = _gather_kernel(lab2d, weight)
    return out.reshape(BATCH, HIST, DIM)


# speedup vs baseline: 1.1062x; 1.0007x over previous
import functools

import jax
import jax.numpy as jnp
from jax import lax
from jax.experimental import pallas as pl
from jax.experimental.pallas import tpu as pltpu
from jax.experimental.pallas import tpu_sc as plsc

NUM_EMB = 1_000_000
DIM = 32
BATCH = 16384
HIST = 50
B = BATCH * HIST

NC = 2
NS = 16
NW = NC * NS
B_PER_W = B // NW

BLK = 1024
N_BLK = B_PER_W // BLK
N_PAIR = (N_BLK - 1) // 2

_mesh = plsc.VectorSubcoreMesh(core_axis_name="c", subcore_axis_name="s")


@functools.partial(
    pl.kernel,
    mesh=_mesh,
    out_type=jax.ShapeDtypeStruct((B, DIM), jnp.float32),
    scratch_types=[
        pltpu.VMEM((2, BLK), jnp.int32),
        pltpu.VMEM((2, BLK, DIM), jnp.float32),
        pltpu.SemaphoreType.DMA,
        pltpu.SemaphoreType.DMA,
        pltpu.SemaphoreType.DMA,
        pltpu.SemaphoreType.DMA,
        pltpu.SemaphoreType.DMA,
    ],
    compiler_params=pltpu.CompilerParams(use_tc_tiling_on_sc=False),
)
def _gather_kernel(lab_hbm, w_hbm, out_hbm, idx_v, rows_v, sem_i,
                   sem_g0, sem_g1, sem_o0, sem_o1):
    wid = lax.axis_index("s") * NC + lax.axis_index("c")
    row0 = wid * B_PER_W
    sem_g = (sem_g0, sem_g1)
    sem_o = (sem_o0, sem_o1)

    def stage_idx(b, u):
        pltpu.async_copy(lab_hbm.at[pl.ds(row0 + b * BLK, BLK)],
                         idx_v.at[u], sem_i).wait()

    def fire_gather(u):
        pltpu.async_copy(w_hbm.at[idx_v.at[u]], rows_v.at[u], sem_g[u])

    def wait_gather(u):
        pltpu.make_async_copy(w_hbm.at[idx_v.at[u]], rows_v.at[u],
                              sem_g[u]).wait()

    def fire_out(b, u):
        pltpu.async_copy(rows_v.at[u],
                         out_hbm.at[pl.ds(row0 + b * BLK, BLK)], sem_o[u])

    def wait_out(b, u):
        pltpu.make_async_copy(rows_v.at[u],
                              out_hbm.at[pl.ds(row0 + b * BLK, BLK)],
                              sem_o[u]).wait()

    stage_idx(0, 0)
    fire_gather(0)

    def body(p, carry):
        for u in (0, 1):
            v = 1 - u
            b = 2 * p + u
            wait_gather(u)
            fire_out(b, u)

            @pl.when(b + 1 < N_BLK)
            def _():
                stage_idx(b + 1, v)

            @pl.when(b >= 1)
            def _():
                wait_out(b - 1, v)

            @pl.when(b + 1 < N_BLK)
            def _():
                fire_gather(v)

        return carry

    lax.fori_loop(0, N_PAIR, body, 0)
    last = N_BLK - 1
    wait_gather(last % 2)
    fire_out(last, last % 2)
    wait_out(last - 1, 1 - (last % 2))
    wait_out(last, last % 2)


def kernel(labels, weight):
    lab1d = labels.astype(jnp.int32).reshape(B)
    out = _gather_kernel(lab1d, weight)
    return out.reshape(BATCH, HIST, DIM)


# speedup vs baseline: 1.7114x; 1.5472x over previous
import functools

import jax
import jax.numpy as jnp
from jax import lax
from jax.experimental import pallas as pl
from jax.experimental.pallas import tpu as pltpu
from jax.experimental.pallas import tpu_sc as plsc

NUM_EMB = 1_000_000
DIM = 32
BATCH = 16384
HIST = 50

NC = 2
NS = 16
NW = NC * NS
ROWS_PER_W = BATCH // NW

RB = 8
N_BLK = ROWS_PER_W // RB
N_PAIR = N_BLK // 2

_mesh = plsc.VectorSubcoreMesh(core_axis_name="c", subcore_axis_name="s")


@functools.partial(
    pl.kernel,
    mesh=_mesh,
    out_type=jax.ShapeDtypeStruct((BATCH, HIST, DIM), jnp.float32),
    scratch_types=[
        pltpu.VMEM((2, RB, HIST), jnp.int32),
        pltpu.VMEM((2, RB, HIST, DIM), jnp.float32),
        pltpu.SemaphoreType.DMA,
        pltpu.SemaphoreType.DMA,
        pltpu.SemaphoreType.DMA,
        pltpu.SemaphoreType.DMA,
        pltpu.SemaphoreType.DMA,
    ],
    compiler_params=pltpu.CompilerParams(use_tc_tiling_on_sc=False),
)
def _gather_kernel(lab_hbm, w_hbm, out_hbm, idx_v, rows_v, sem_i,
                   sem_g0, sem_g1, sem_o0, sem_o1):
    wid = lax.axis_index("s") * NC + lax.axis_index("c")
    row0 = wid * ROWS_PER_W
    sem_g = (sem_g0, sem_g1)
    sem_o = (sem_o0, sem_o1)

    def stage_idx(b, u):
        pltpu.async_copy(lab_hbm.at[pl.ds(row0 + b * RB, RB)],
                         idx_v.at[u], sem_i).wait()

    def fire_gathers(u):
        for j in range(RB):
            pltpu.async_copy(w_hbm.at[idx_v.at[u, j]], rows_v.at[u, j],
                             sem_g[u])

    def wait_gathers(u):
        for j in range(RB):
            pltpu.make_async_copy(w_hbm.at[idx_v.at[u, j]], rows_v.at[u, j],
                                  sem_g[u]).wait()

    def fire_out(b, u):
        pltpu.async_copy(rows_v.at[u],
                         out_hbm.at[pl.ds(row0 + b * RB, RB)], sem_o[u])

    def wait_out(b, u):
        pltpu.make_async_copy(rows_v.at[u],
                              out_hbm.at[pl.ds(row0 + b * RB, RB)],
                              sem_o[u]).wait()

    stage_idx(0, 0)
    fire_gathers(0)

    def body(p, carry):
        for u in (0, 1):
            v = 1 - u
            b = 2 * p + u
            wait_gathers(u)
            fire_out(b, u)

            @pl.when(b + 1 < N_BLK)
            def _():
                stage_idx(b + 1, v)

            @pl.when(b >= 1)
            def _():
                wait_out(b - 1, v)

            @pl.when(b + 1 < N_BLK)
            def _():
                fire_gathers(v)

        return carry

    lax.fori_loop(0, N_PAIR, body, 0)
    wait_out(N_BLK - 1, 1)


def kernel(labels, weight):
    return _gather_kernel(labels.astype(jnp.int32), weight)
